# baseline (speedup 1.0000x reference)
import functools

import jax
import jax.numpy as jnp
from jax import lax
from jax.experimental import pallas as pl
from jax.experimental.pallas import tpu as pltpu
from jax.experimental.pallas import tpu_sc as plsc

N = 16384
WIN = 5
EMB = 64
NCLASS = 9

_NUM_IDX = N * WIN

_NC = 2
_NS = 16
_NW = _NC * _NS
_S_W = N // _NW
_R_W = _S_W * WIN
_C_S = 64
_C_R = _C_S * WIN
_NCHUNK = _S_W // _C_S


def _extract_chunk(pairbuf, hoff_v, outb, c):
    iota = lax.iota(jnp.int32, 16)
    for g in range(_C_S // 16):
        srow = g * 16 + iota
        entries = []
        for w in range(WIN):
            prow = srow * WIN + w
            hoffv = plsc.load_gather(hoff_v, [c * _C_R + prow])
            entries.append((prow, hoffv, srow, w * EMB))

        def body(k, _, entries=entries):
            for prow, hoffv, srow, cb in entries:
                v = plsc.load_gather(pairbuf, [prow, hoffv + k])
                plsc.store_scatter(outb, [srow, cb + k + jnp.zeros((16,), jnp.int32)], v)
            return 0

        lax.fori_loop(0, EMB, body, 0)


def _sc_gather(pidx, hoff, table2):
    mesh = plsc.VectorSubcoreMesh(core_axis_name="c", subcore_axis_name="s")

    @functools.partial(
        pl.kernel,
        out_type=jax.ShapeDtypeStruct((N, WIN * EMB), jnp.float32),
        mesh=mesh,
        scratch_types=[
            pltpu.VMEM((_R_W,), jnp.int32),
            pltpu.VMEM((_R_W,), jnp.int32),
            pltpu.VMEM((_C_R, 128), jnp.float32),
            pltpu.VMEM((_C_R, 128), jnp.float32),
            pltpu.VMEM((_C_S, WIN * EMB), jnp.float32),
            pltpu.SemaphoreType.DMA,
            pltpu.SemaphoreType.DMA,
        ],
        compiler_params=pltpu.CompilerParams(needs_layout_passes=False),
    )
    def gather_kernel(pidx_hbm, hoff_hbm, table2_hbm, out_hbm,
                      pidx_v, hoff_v, pair0, pair1, outb, sem0, sem1):
        wid = lax.axis_index("s") * _NC + lax.axis_index("c")
        rowbase = wid * _R_W
        sbase = wid * _S_W
        pltpu.sync_copy(pidx_hbm.at[pl.ds(rowbase, _R_W)], pidx_v)
        pltpu.sync_copy(hoff_hbm.at[pl.ds(rowbase, _R_W)], hoff_v)
        bufs = (pair0, pair1)
        sems = (sem0, sem1)
        copies = [pltpu.async_copy(
            table2_hbm.at[pidx_v.at[pl.ds(0, _C_R)]], pair0, sem0)]
        for c in range(_NCHUNK):
            if c + 1 < _NCHUNK:
                copies.append(pltpu.async_copy(
                    table2_hbm.at[pidx_v.at[pl.ds((c + 1) * _C_R, _C_R)]],
                    bufs[(c + 1) % 2],
                    sems[(c + 1) % 2],
                ))
            copies[c].wait()
            _extract_chunk(bufs[c % 2], hoff_v, outb, c)
            pltpu.sync_copy(outb, out_hbm.at[pl.ds(sbase + c * _C_S, _C_S)])

    return gather_kernel(pidx, hoff, table2)


def _tc_linear(flat, wt, b2d):
    bn = 4096

    def mm_kernel(flat_ref, wt_ref, b_ref, out_ref):
        out_ref[...] = (
            jnp.dot(flat_ref[...], wt_ref[...], preferred_element_type=jnp.float32)
            + b_ref[...]
        )

    return pl.pallas_call(
        mm_kernel,
        grid=(N // bn,),
        in_specs=[
            pl.BlockSpec((bn, WIN * EMB), lambda i: (i, 0)),
            pl.BlockSpec((WIN * EMB, NCLASS), lambda i: (0, 0)),
            pl.BlockSpec((1, NCLASS), lambda i: (0, 0)),
        ],
        out_specs=pl.BlockSpec((bn, NCLASS), lambda i: (i, 0)),
        out_shape=jax.ShapeDtypeStruct((N, NCLASS), jnp.float32),
    )(flat, wt, b2d)


def kernel(x, table, W, b):
    idx = x.reshape(-1).astype(jnp.int32)
    pidx = idx >> 1
    hoff = (idx & 1) << 6
    table2 = table.reshape(500000, 128)
    flat = _sc_gather(pidx, hoff, table2)
    out = _tc_linear(flat, W.T, b.reshape(1, NCLASS))
    return out
